# Initial kernel scaffold; baseline (speedup 1.0000x reference)
import jax
import jax.numpy as jnp
from jax.experimental import pallas as pl


def kernel(inputs):
    raise NotImplementedError("write your pallas kernel here")


# speedup vs baseline: 7.4365x; 7.4365x over previous
import jax
import jax.numpy as jnp
from jax.experimental import pallas as pl

N_TOKENS = 8192
D_MODEL = 768
PATHS = 8
BLOCK = 1024


def _dispatch_body(in_ref, *out_refs):
    row = in_ref[...]
    gate = jnp.float32(1.0)
    out_refs[0][...] = row * gate
    zeros = jnp.zeros_like(row)
    for r in out_refs[1:]:
        r[...] = zeros


def kernel(inputs):
    n, d = inputs.shape
    grid = (n // BLOCK,)
    spec = pl.BlockSpec((BLOCK, d), lambda i: (i, 0))
    out_shape = tuple(
        jax.ShapeDtypeStruct((n, d), inputs.dtype) for _ in range(PATHS)
    )
    return pl.pallas_call(
        _dispatch_body,
        grid=grid,
        in_specs=[spec],
        out_specs=tuple(spec for _ in range(PATHS)),
        out_shape=out_shape,
    )(inputs)
